# baseline (speedup 1.0000x reference)
import functools

import jax
import jax.numpy as jnp
from jax import lax
from jax.experimental import pallas as pl
from jax.experimental.pallas import tpu as pltpu
from jax.experimental.pallas import tpu_sc as plsc

G = 16
N2 = 32
T = G * G
C = 3
N2K = C * N2 * N2
B = 64
BS = 16
NB = 8

NC, NS = 2, 16
NW = NC * NS
MAXFIX = 256


def _sc_body(x_hbm, out_hbm, rowbuf, gat_sem, scat_sem):
    chunks = (BS * T) // (NW * G)
    wid = lax.axis_index("s") * NC + lax.axis_index("c")

    def gather_chunk(g, slot):
        band = wid * chunks + g
        b = lax.div(band, G)
        g1 = lax.rem(band, G)
        for i in range(G):
            pltpu.make_async_copy(
                x_hbm.at[b, :, pl.ds(g1 * N2, N2), pl.ds(i * N2, N2)],
                rowbuf.at[slot, i],
                gat_sem.at[slot],
            ).start()

    def wait_gathers(slot):
        pltpu.make_async_copy(
            out_hbm.at[pl.ds(0, G)],
            rowbuf.at[slot],
            gat_sem.at[slot],
        ).wait()

    def scatter_chunk(h, slot):
        band = wid * chunks + h
        pltpu.async_copy(
            rowbuf.at[slot],
            out_hbm.at[pl.ds(band * G, G)],
            scat_sem.at[slot],
        )

    def wait_scatter(slot):
        pltpu.make_async_copy(
            out_hbm.at[pl.ds(0, G)],
            rowbuf.at[slot],
            scat_sem.at[slot],
        ).wait()

    def loop_body(g, carry):
        slot = lax.rem(g, 2)

        @pl.when(g < chunks)
        def _issue():
            @pl.when(g >= 2)
            def _reuse():
                wait_scatter(slot)

            gather_chunk(g, slot)

        @pl.when(g >= 1)
        def _process():
            h = g - 1
            sloth = lax.rem(h, 2)
            wait_gathers(sloth)
            scatter_chunk(h, sloth)

        return carry

    lax.fori_loop(0, chunks + 1, loop_body, 0)
    wait_scatter(0)
    wait_scatter(1)


def _tc_kernel(x_ref, m_ref, repl_ref, out_ref):
    nb = x_ref.shape[2] // N2
    x = x_ref[0]
    y = x.reshape(C, nb, N2, G, N2).transpose(1, 3, 0, 2, 4).reshape(nb * G, N2K)
    m = m_ref[0, 0, 0, :]
    repl = repl_ref[0]
    out_ref[0] = jnp.where(m[:, None] > 0.5, repl[None, :], y)


def _merge_kernel(sc_ref, _, out_ref, sem):
    cp = pltpu.make_async_copy(sc_ref, out_ref.at[pl.ds(0, BS * T), :], sem)
    cp.start()
    cp.wait()


def _fix_kernel(rows_ref, cnt_ref, repl_ref, _, out_ref, sem):
    cnt = cnt_ref[0, 0]

    def start(k, carry):
        pltpu.make_async_copy(
            repl_ref, out_ref.at[pl.ds(rows_ref[0, k], 1), :], sem
        ).start()
        return carry

    def drain(k, carry):
        pltpu.make_async_copy(
            repl_ref, out_ref.at[pl.ds(0, 1), :], sem
        ).wait()
        return carry

    lax.fori_loop(0, cnt, start, 0)
    lax.fori_loop(0, cnt, drain, 0)


def kernel(X):
    b = X.shape[0]
    bt = b - BS
    k1, k2 = jax.random.split(jax.random.key(1))
    idx = jax.random.bernoulli(k1, 1.0 / T, (b * T,))
    repl = jnp.tanh(jax.random.normal(k2, (N2K,), dtype=jnp.float32))

    mesh = plsc.VectorSubcoreMesh(
        core_axis_name="c", subcore_axis_name="s",
        num_cores=NC, num_subcores=NS,
    )
    sc_fn = functools.partial(
        pl.kernel,
        out_type=jax.ShapeDtypeStruct((BS * T, C, N2, N2), jnp.float32),
        mesh=mesh,
        scratch_types=[
            pltpu.VMEM((2, G, C, N2, N2), jnp.float32),
            pltpu.SemaphoreType.DMA((2,)),
            pltpu.SemaphoreType.DMA((2,)),
        ],
        compiler_params=pltpu.CompilerParams(use_tc_tiling_on_sc=False),
    )(_sc_body)
    sc_part = sc_fn(X[:BS]).reshape(BS * T, N2K)

    m4 = idx.reshape(b, G // NB, 1, NB * G).astype(jnp.float32)
    out_full = pl.pallas_call(
        _tc_kernel,
        grid=(bt, G // NB),
        in_specs=[
            pl.BlockSpec((1, C, NB * N2, G * N2), lambda i, j: (i + BS, 0, j, 0)),
            pl.BlockSpec((1, 1, 1, NB * G), lambda i, j: (i + BS, j, 0, 0)),
            pl.BlockSpec((1, N2K), lambda i, j: (0, 0)),
        ],
        out_specs=pl.BlockSpec((1, NB * G, N2K), lambda i, j: (i + BS, j, 0)),
        out_shape=jax.ShapeDtypeStruct((b, T, N2K), jnp.float32),
    )(X, m4, repl.reshape(1, N2K))

    merged = pl.pallas_call(
        _merge_kernel,
        in_specs=[
            pl.BlockSpec(memory_space=pl.ANY),
            pl.BlockSpec(memory_space=pl.ANY),
        ],
        out_specs=pl.BlockSpec(memory_space=pl.ANY),
        out_shape=jax.ShapeDtypeStruct((b * T, N2K), jnp.float32),
        scratch_shapes=[pltpu.SemaphoreType.DMA],
        input_output_aliases={1: 0},
    )(sc_part, out_full.reshape(b * T, N2K))

    sc_rows = jnp.nonzero(idx[:BS * T], size=MAXFIX, fill_value=0)[0]
    rows2 = sc_rows.astype(jnp.int32).reshape(1, MAXFIX)
    cnt2 = jnp.sum(idx[:BS * T]).astype(jnp.int32).reshape(1, 1)

    out = pl.pallas_call(
        _fix_kernel,
        in_specs=[
            pl.BlockSpec(memory_space=pltpu.MemorySpace.SMEM),
            pl.BlockSpec(memory_space=pltpu.MemorySpace.SMEM),
            pl.BlockSpec(memory_space=pltpu.MemorySpace.VMEM),
            pl.BlockSpec(memory_space=pl.ANY),
        ],
        out_specs=pl.BlockSpec(memory_space=pl.ANY),
        out_shape=jax.ShapeDtypeStruct((b * T, N2K), jnp.float32),
        scratch_shapes=[pltpu.SemaphoreType.DMA],
        input_output_aliases={3: 0},
    )(rows2, cnt2, repl.reshape(1, N2K), merged)

    return out.reshape(b, T, N2K), idx


# speedup vs baseline: 3.1051x; 3.1051x over previous
import functools

import jax
import jax.numpy as jnp
from jax import lax
from jax.experimental import pallas as pl
from jax.experimental.pallas import tpu as pltpu
from jax.experimental.pallas import tpu_sc as plsc

G = 16
N2 = 32
T = G * G
C = 3
N2K = C * N2 * N2
B = 64
BS = 16
NB = 8

NC, NS = 2, 16
NW = NC * NS
MAXFIX = 256


def _sc_body(x_hbm, out_hbm, rowbuf, gat_sem, scat_sem):
    chunks = (BS * T) // (NW * G)
    wid = lax.axis_index("s") * NC + lax.axis_index("c")

    def gather_chunk(g, slot):
        band = wid * chunks + g
        b = lax.div(band, G)
        g1 = lax.rem(band, G)
        for i in range(G):
            pltpu.make_async_copy(
                x_hbm.at[b, :, pl.ds(g1 * N2, N2), pl.ds(i * N2, N2)],
                rowbuf.at[slot, i],
                gat_sem.at[slot],
            ).start()

    def wait_gathers(slot):
        pltpu.make_async_copy(
            out_hbm.at[pl.ds(0, G)],
            rowbuf.at[slot],
            gat_sem.at[slot],
        ).wait()

    def scatter_chunk(h, slot):
        band = wid * chunks + h
        pltpu.async_copy(
            rowbuf.at[slot],
            out_hbm.at[pl.ds(band * G, G)],
            scat_sem.at[slot],
        )

    def wait_scatter(slot):
        pltpu.make_async_copy(
            out_hbm.at[pl.ds(0, G)],
            rowbuf.at[slot],
            scat_sem.at[slot],
        ).wait()

    def loop_body(g, carry):
        slot = lax.rem(g, 2)

        @pl.when(g < chunks)
        def _issue():
            @pl.when(g >= 2)
            def _reuse():
                wait_scatter(slot)

            gather_chunk(g, slot)

        @pl.when(g >= 1)
        def _process():
            h = g - 1
            sloth = lax.rem(h, 2)
            wait_gathers(sloth)
            scatter_chunk(h, sloth)

        return carry

    lax.fori_loop(0, chunks + 1, loop_body, 0)
    wait_scatter(0)
    wait_scatter(1)


def _tc_kernel(x_ref, m_ref, repl_ref, out_ref):
    nb = x_ref.shape[2] // N2
    x = x_ref[0]
    y = x.reshape(C, nb, N2, G, N2).transpose(1, 3, 0, 2, 4).reshape(nb * G, N2K)
    m = m_ref[0, 0, 0, :]
    repl = repl_ref[0]
    out_ref[0] = jnp.where(m[:, None] > 0.5, repl[None, :], y)


def _merge_kernel(sc_ref, _, out_ref):
    out_ref[...] = sc_ref[...]


def _fix_kernel(rows_ref, cnt_ref, repl_ref, _, out_ref, sem):
    cnt = cnt_ref[0, 0]

    def start(k, carry):
        pltpu.make_async_copy(
            repl_ref, out_ref.at[pl.ds(rows_ref[0, k], 1), :], sem
        ).start()
        return carry

    def drain(k, carry):
        pltpu.make_async_copy(
            repl_ref, out_ref.at[pl.ds(0, 1), :], sem
        ).wait()
        return carry

    lax.fori_loop(0, cnt, start, 0)
    lax.fori_loop(0, cnt, drain, 0)


def kernel(X):
    b = X.shape[0]
    bt = b - BS
    k1, k2 = jax.random.split(jax.random.key(1))
    idx = jax.random.bernoulli(k1, 1.0 / T, (b * T,))
    repl = jnp.tanh(jax.random.normal(k2, (N2K,), dtype=jnp.float32))

    mesh = plsc.VectorSubcoreMesh(
        core_axis_name="c", subcore_axis_name="s",
        num_cores=NC, num_subcores=NS,
    )
    sc_fn = functools.partial(
        pl.kernel,
        out_type=jax.ShapeDtypeStruct((BS * T, C, N2, N2), jnp.float32),
        mesh=mesh,
        scratch_types=[
            pltpu.VMEM((2, G, C, N2, N2), jnp.float32),
            pltpu.SemaphoreType.DMA((2,)),
            pltpu.SemaphoreType.DMA((2,)),
        ],
        compiler_params=pltpu.CompilerParams(use_tc_tiling_on_sc=False),
    )(_sc_body)
    sc_part = sc_fn(X[:BS]).reshape(BS * T, N2K)

    m4 = idx.reshape(b, G // NB, 1, NB * G).astype(jnp.float32)
    out_full = pl.pallas_call(
        _tc_kernel,
        grid=(bt, G // NB),
        in_specs=[
            pl.BlockSpec((1, C, NB * N2, G * N2), lambda i, j: (i + BS, 0, j, 0)),
            pl.BlockSpec((1, 1, 1, NB * G), lambda i, j: (i + BS, j, 0, 0)),
            pl.BlockSpec((1, N2K), lambda i, j: (0, 0)),
        ],
        out_specs=pl.BlockSpec((1, NB * G, N2K), lambda i, j: (i + BS, j, 0)),
        out_shape=jax.ShapeDtypeStruct((b, T, N2K), jnp.float32),
    )(X, m4, repl.reshape(1, N2K))

    merged = pl.pallas_call(
        _merge_kernel,
        grid=(BS * T // 1024,),
        in_specs=[
            pl.BlockSpec((1024, N2K), lambda i: (i, 0)),
            pl.BlockSpec(memory_space=pl.ANY),
        ],
        out_specs=pl.BlockSpec((1024, N2K), lambda i: (i, 0)),
        out_shape=jax.ShapeDtypeStruct((b * T, N2K), jnp.float32),
        input_output_aliases={1: 0},
    )(sc_part, out_full.reshape(b * T, N2K))

    sc_rows = jnp.nonzero(idx[:BS * T], size=MAXFIX, fill_value=0)[0]
    rows2 = sc_rows.astype(jnp.int32).reshape(1, MAXFIX)
    cnt2 = jnp.sum(idx[:BS * T]).astype(jnp.int32).reshape(1, 1)

    out = pl.pallas_call(
        _fix_kernel,
        in_specs=[
            pl.BlockSpec(memory_space=pltpu.MemorySpace.SMEM),
            pl.BlockSpec(memory_space=pltpu.MemorySpace.SMEM),
            pl.BlockSpec(memory_space=pltpu.MemorySpace.VMEM),
            pl.BlockSpec(memory_space=pl.ANY),
        ],
        out_specs=pl.BlockSpec(memory_space=pl.ANY),
        out_shape=jax.ShapeDtypeStruct((b * T, N2K), jnp.float32),
        scratch_shapes=[pltpu.SemaphoreType.DMA],
        input_output_aliases={3: 0},
    )(rows2, cnt2, repl.reshape(1, N2K), merged)

    return out.reshape(b, T, N2K), idx


# speedup vs baseline: 4.4241x; 1.4248x over previous
import jax
import jax.numpy as jnp
from jax.experimental import pallas as pl

G = 16
N2 = 32
T = G * G
C = 3
N2K = C * N2 * N2


def _patch_kernel(x_ref, m_ref, repl_ref, out_ref):
    nb = x_ref.shape[2] // N2
    x = x_ref[0]
    y = x.reshape(C, nb, N2, G, N2).transpose(1, 3, 0, 2, 4).reshape(nb * G, N2K)
    m = m_ref[0, 0, 0, :]
    repl = repl_ref[0]
    out_ref[0] = jnp.where(m[:, None] > 0.5, repl[None, :], y)


def kernel(X):
    b = X.shape[0]
    k1, k2 = jax.random.split(jax.random.key(1))
    idx = jax.random.bernoulli(k1, 1.0 / T, (b * T,))
    repl = jnp.tanh(jax.random.normal(k2, (N2K,), dtype=jnp.float32))

    NB = 8
    m4 = idx.reshape(b, G // NB, 1, NB * G).astype(jnp.float32)
    repl2 = repl.reshape(1, N2K)

    out = pl.pallas_call(
        _patch_kernel,
        grid=(b, G // NB),
        in_specs=[
            pl.BlockSpec((1, C, NB * N2, G * N2), lambda i, j: (i, 0, j, 0)),
            pl.BlockSpec((1, 1, 1, NB * G), lambda i, j: (i, j, 0, 0)),
            pl.BlockSpec((1, N2K), lambda i, j: (0, 0)),
        ],
        out_specs=pl.BlockSpec((1, NB * G, N2K), lambda i, j: (i, j, 0)),
        out_shape=jax.ShapeDtypeStruct((b, T, N2K), jnp.float32),
    )(X, m4, repl2)

    return out, idx


# speedup vs baseline: 4.4348x; 1.0024x over previous
import jax
import jax.numpy as jnp
from jax import lax
from jax.experimental import pallas as pl
from jax.experimental.pallas import tpu as pltpu

G = 16
N2 = 32
T = G * G
C = 3
N2K = C * N2 * N2
NB = 8
MAXFIX = 256


def _patch_kernel(x_ref, out_ref):
    nb = x_ref.shape[2] // N2
    x = x_ref[0]
    y = x.reshape(C, nb, N2, G, N2).transpose(1, 3, 0, 2, 4).reshape(nb * G, N2K)
    out_ref[0] = y


def _fix_kernel(rows_ref, cnt_ref, repl_ref, _, out_ref, sem):
    cnt = cnt_ref[0, 0]

    def start(k, carry):
        pltpu.make_async_copy(
            repl_ref, out_ref.at[pl.ds(rows_ref[0, k], 1), :], sem
        ).start()
        return carry

    def drain(k, carry):
        pltpu.make_async_copy(
            repl_ref, out_ref.at[pl.ds(0, 1), :], sem
        ).wait()
        return carry

    lax.fori_loop(0, cnt, start, 0)
    lax.fori_loop(0, cnt, drain, 0)


def kernel(X):
    b = X.shape[0]
    k1, k2 = jax.random.split(jax.random.key(1))
    idx = jax.random.bernoulli(k1, 1.0 / T, (b * T,))
    repl = jnp.tanh(jax.random.normal(k2, (N2K,), dtype=jnp.float32))

    patched = pl.pallas_call(
        _patch_kernel,
        grid=(b, G // NB),
        in_specs=[
            pl.BlockSpec((1, C, NB * N2, G * N2), lambda i, j: (i, 0, j, 0)),
        ],
        out_specs=pl.BlockSpec((1, NB * G, N2K), lambda i, j: (i, j, 0)),
        out_shape=jax.ShapeDtypeStruct((b, T, N2K), jnp.float32),
    )(X)

    rows = jnp.nonzero(idx, size=MAXFIX, fill_value=0)[0]
    rows2 = rows.astype(jnp.int32).reshape(1, MAXFIX)
    cnt2 = jnp.sum(idx).astype(jnp.int32).reshape(1, 1)

    out = pl.pallas_call(
        _fix_kernel,
        in_specs=[
            pl.BlockSpec(memory_space=pltpu.MemorySpace.SMEM),
            pl.BlockSpec(memory_space=pltpu.MemorySpace.SMEM),
            pl.BlockSpec(memory_space=pltpu.MemorySpace.VMEM),
            pl.BlockSpec(memory_space=pl.ANY),
        ],
        out_specs=pl.BlockSpec(memory_space=pl.ANY),
        out_shape=jax.ShapeDtypeStruct((b * T, N2K), jnp.float32),
        scratch_shapes=[pltpu.SemaphoreType.DMA],
        input_output_aliases={3: 0},
    )(rows2, cnt2, repl.reshape(1, N2K), patched.reshape(b * T, N2K))

    return out.reshape(b, T, N2K), idx
